# baseline (speedup 1.0000x reference)
import functools

import jax
import jax.numpy as jnp
from jax import lax
from jax.experimental import pallas as pl
from jax.experimental.pallas import tpu as pltpu
from jax.experimental.pallas import tpu_sc as plsc


def _build_warp(B, H, W, C):
    N = B * H * W
    info = plsc.get_sparse_core_info()
    NC, NS, L = info.num_cores, info.num_subcores, info.num_lanes
    NW = NC * NS
    assert N % NW == 0 and C % L == 0 and L == 16
    NPT = N // NW
    P = 8
    D = 4
    assert NPT % (P * D) == 0
    NCH = NPT // P
    ROWS = 4 * P
    HW = H * W

    mesh = plsc.VectorSubcoreMesh(core_axis_name="c", subcore_axis_name="s")

    @functools.partial(
        pl.kernel,
        mesh=mesh,
        out_type=jax.ShapeDtypeStruct((N, C), jnp.float32),
        scratch_types=[
            pltpu.VMEM((D, 2 * P), jnp.float32),
            pltpu.VMEM((D, ROWS), jnp.int32),
            pltpu.VMEM((D * 4 * P * 16,), jnp.float32),
            pltpu.VMEM((D, ROWS, C), jnp.float32),
            pltpu.VMEM((D, P, C), jnp.float32),
            [pltpu.SemaphoreType.DMA] * D,
            [pltpu.SemaphoreType.DMA] * D,
            [pltpu.SemaphoreType.DMA] * D,
        ],
        compiler_params=pltpu.CompilerParams(needs_layout_passes=False),
    )
    def warp(x_hbm, fl_hbm, out_hbm,
             flc, idxc, wexp, rows_v, out_v, fsem, gsem, osem):
        wid = lax.axis_index("s") * NC + lax.axis_index("c")
        base = wid * NPT
        base2 = base * 2

        lanes = lax.iota(jnp.int32, L)
        lanem = lanes & (P - 1)
        evens = lanem * 2
        odds = evens + 1
        pmask = lanes < P
        scat = lanem * L
        WPB = 4 * P * L

        def fstart(ch, j):
            pltpu.async_copy(
                fl_hbm.at[pl.ds(base2 + ch * (2 * P), 2 * P)],
                flc.at[j], fsem[j])

        def fwait(ch, j):
            pltpu.make_async_copy(
                fl_hbm.at[pl.ds(base2 + ch * (2 * P), 2 * P)],
                flc.at[j], fsem[j]).wait()

        def gstart(ch, j):
            pltpu.async_copy(x_hbm.at[idxc.at[j]], rows_v.at[j], gsem[j])

        def gwait(ch, j):
            pltpu.make_async_copy(
                x_hbm.at[idxc.at[j]], rows_v.at[j], gsem[j]).wait()

        def ostart(ch, j):
            pltpu.async_copy(
                out_v.at[j], out_hbm.at[pl.ds(base + ch * P, P)], osem[j])

        def owait(ch, j):
            pltpu.make_async_copy(
                out_v.at[j], out_hbm.at[pl.ds(base + ch * P, P)],
                osem[j]).wait()

        def idxw(ch, j):
            fxv = plsc.load_gather(flc.at[j], [evens])
            fyv = plsc.load_gather(flc.at[j], [odds])
            fxv = jnp.minimum(jnp.maximum(fxv, -512.0), 512.0)
            fyv = jnp.minimum(jnp.maximum(fyv, -512.0), 512.0)
            px = base + ch * P + lanem
            b = jnp.where(px >= HW, 1, 0)
            r = px - b * HW
            iy = r // W
            ix = r - iy * W
            fx0 = fxv.astype(jnp.int32)
            fx0 = fx0 - jnp.where(fx0.astype(jnp.float32) > fxv, 1, 0)
            fy0 = fyv.astype(jnp.int32)
            fy0 = fy0 - jnp.where(fy0.astype(jnp.float32) > fyv, 1, 0)
            gx0 = jnp.minimum(jnp.maximum(ix + fx0, 0), W - 1)
            gx1 = jnp.minimum(jnp.maximum(ix + fx0 + 1, 0), W - 1)
            gy0 = jnp.minimum(jnp.maximum(iy + fy0, 0), H - 1)
            gy1 = jnp.minimum(jnp.maximum(iy + fy0 + 1, 0), H - 1)
            row0 = b * HW + gy0 * W
            row1 = b * HW + gy1 * W
            idxs = (row0 + gx0, row0 + gx1, row1 + gx0, row1 + gx1)
            jv = jnp.full((L,), j, jnp.int32)
            for k in range(4):
                plsc.store_scatter(
                    idxc, [jv, k * P + lanem], idxs[k], mask=pmask)
            cx1 = fxv - fx0.astype(jnp.float32)
            cx0 = 1.0 - cx1
            cy1 = fyv - fy0.astype(jnp.float32)
            cy0 = 1.0 - cy1
            ws = (cy0 * cx0, cy0 * cx1, cy1 * cx0, cy1 * cx1)
            for k in range(4):
                for c in range(L):
                    plsc.store_scatter(
                        wexp, [scat + (j * WPB + k * P * L + c)],
                        ws[k], mask=pmask)

        def combine(ch, j):
            wb = j * WPB

            def body_px(i, carry):
                b00 = wexp[pl.ds(wb + 0 * P * L + i * L, L)]
                b01 = wexp[pl.ds(wb + 1 * P * L + i * L, L)]
                b10 = wexp[pl.ds(wb + 2 * P * L + i * L, L)]
                b11 = wexp[pl.ds(wb + 3 * P * L + i * L, L)]
                for c in range(C // L):
                    s = pl.ds(c * L, L)
                    out_v[j, i, s] = (
                        b00 * rows_v[j, i, s]
                        + b01 * rows_v[j, P + i, s]
                        + b10 * rows_v[j, 2 * P + i, s]
                        + b11 * rows_v[j, 3 * P + i, s])
                return carry

            lax.fori_loop(0, P, body_px, 0)

        for j in range(D):
            fstart(j, j)
        for j in range(2):
            fwait(j, j)
            idxw(j, j)
            gstart(j, j)

        def step(td, carry):
            for r in range(D):
                ch = td * D + r

                @pl.when(ch + D < NCH)
                def _():
                    fstart(ch + D, r)

                @pl.when(ch + 2 < NCH)
                def _():
                    fwait(ch + 2, (r + 2) % D)
                    idxw(ch + 2, (r + 2) % D)
                    gstart(ch + 2, (r + 2) % D)

                gwait(ch, r)

                @pl.when(ch >= D)
                def _():
                    owait(ch - D, r)

                combine(ch, r)
                ostart(ch, r)
            return carry

        lax.fori_loop(0, NCH // D, step, 0)
        for j in range(D):
            owait(NCH - D + j, j)

    return warp


def kernel(x, flow):
    B, H, W, C = x.shape
    warp = _build_warp(B, H, W, C)
    xt = x.reshape(B * H * W, C)
    fl = flow.reshape(B * H * W * 2)
    out = warp(xt, fl)
    return out.reshape(B, H, W, C)
